# baseline (speedup 1.0000x reference)
import functools

import jax
import jax.numpy as jnp
from jax import lax
from jax.experimental import pallas as pl
from jax.experimental.pallas import tpu as pltpu

_P = 2


def _block_kernel(x_ref, w1_ref, b1_ref, wd_ref, sd_ref, bd_ref,
                  w2_ref, b2_ref, o_ref, pad_ref, *, c1):
    P, D, H, W, C = x_ref.shape
    M = D * H * W
    ch = C - c1
    Cd = P * ch
    x2 = jnp.concatenate([x_ref[p][..., c1:] for p in range(P)],
                         axis=-1).reshape(M, Cd)
    xp = jnp.concatenate([x_ref[p][..., :c1] for p in range(P)],
                         axis=-1).reshape(M, P * c1)

    y = jnp.dot(x2.astype(jnp.bfloat16), w1_ref[...],
                preferred_element_type=jnp.float32)
    y = jnp.maximum(y + b1_ref[...], 0.0)

    pad_ref[0:1] = jnp.zeros((1, H + 2, W + 2, Cd), jnp.float32)
    pad_ref[D + 1:D + 2] = jnp.zeros((1, H + 2, W + 2, Cd), jnp.float32)
    pad_ref[:, 0:1] = jnp.zeros((D + 2, 1, W + 2, Cd), jnp.float32)
    pad_ref[:, H + 1:H + 2] = jnp.zeros((D + 2, 1, W + 2, Cd), jnp.float32)
    pad_ref[:, :, 0:1] = jnp.zeros((D + 2, H + 2, 1, Cd), jnp.float32)
    pad_ref[:, :, W + 1:W + 2] = jnp.zeros((D + 2, H + 2, 1, Cd), jnp.float32)
    pad_ref[1:D + 1, 1:H + 1, 1:W + 1, :] = y.reshape(D, H, W, Cd)
    acc = jnp.zeros((D, H, W, Cd), jnp.float32)
    for kd in range(3):
        for kh in range(3):
            for kw in range(3):
                idx = kd * 9 + kh * 3 + kw
                tap = pad_ref[kd:kd + D, kh:kh + H, kw:kw + W, :]
                acc = acc + tap * wd_ref[idx:idx + 1, :]
    z = acc * sd_ref[...] + bd_ref[...]

    u = jnp.concatenate([z.reshape(M, Cd).astype(jnp.bfloat16),
                         xp.astype(jnp.bfloat16)], axis=-1)
    v = jnp.dot(u, w2_ref[...], preferred_element_type=jnp.float32)
    g = v + b2_ref[...]
    odd = (lax.broadcasted_iota(jnp.int32, (1, g.shape[1]), 1) % 2) == 1
    out = jnp.where(odd, jnp.maximum(g, 0.0), g)
    for p in range(P):
        o_ref[p] = out[:, p * C:(p + 1) * C].astype(o_ref.dtype)


def _bcast_spec(a):
    return pl.BlockSpec(a.shape, lambda n: (0,) * a.ndim)


def kernel(x, scatter, w1, s1, b1, wd, sd, bd, w2, s2, b2):
    N, C, D, H, W = x.shape
    c1 = scatter.shape[0]
    cm = w1.shape[1]
    oup = w2.shape[1]
    M = D * H * W
    P = _P
    bf = jnp.bfloat16

    xt = jnp.zeros((N, D, H, W, C), x.dtype)

    ch = C - c1
    w1s = w1 * s1
    w1b = jnp.zeros((P * ch, P * cm), jnp.float32)
    for p in range(P):
        w1b = w1b.at[p * ch:(p + 1) * ch, p * cm:(p + 1) * cm].set(w1s)
    w1b = w1b.astype(bf)
    b1p = jnp.tile(b1, (1, P))
    wdp = jnp.tile(wd, (1, P))
    sdp = jnp.tile(sd, (1, P))
    bdp = jnp.tile(bd, (1, P))
    even = (jnp.arange(oup) % 2 == 0).astype(jnp.float32)[None, :]
    w2s = w2 * (s2 + even)
    scs = scatter * (s2 + even)
    Wc = jnp.zeros((P * cm + P * c1, P * oup), jnp.float32)
    for p in range(P):
        Wc = Wc.at[p * cm:(p + 1) * cm, p * oup:(p + 1) * oup].set(w2s)
        Wc = Wc.at[P * cm + p * c1:P * cm + (p + 1) * c1,
                   p * oup:(p + 1) * oup].set(scs)
    Wc = Wc.astype(bf)
    b2c = jnp.tile(b2, (1, P))

    args = (xt, w1b, b1p, wdp, sdp, bdp, Wc, b2c)
    in_specs = [pl.BlockSpec((P, D, H, W, C), lambda n: (n, 0, 0, 0, 0))]
    in_specs += [_bcast_spec(a) for a in args[1:]]
    out = pl.pallas_call(
        functools.partial(_block_kernel, c1=c1),
        out_shape=jax.ShapeDtypeStruct((N, M, oup), x.dtype),
        grid=(N // P,),
        in_specs=in_specs,
        out_specs=pl.BlockSpec((P, M, oup), lambda n: (n, 0, 0)),
        scratch_shapes=[pltpu.VMEM((D + 2, H + 2, W + 2, P * cm), jnp.float32)],
        compiler_params=pltpu.CompilerParams(
            dimension_semantics=("parallel",)),
    )(*args)
    out = out.reshape(N, D, H, W, oup)
    return jnp.transpose(out, (0, 4, 1, 2, 3))


# speedup vs baseline: 1.1591x; 1.1591x over previous
import functools

import jax
import jax.numpy as jnp
from jax import lax
from jax.experimental import pallas as pl
from jax.experimental.pallas import tpu as pltpu

_P = 2


def _block_kernel(x_ref, w1_ref, b1_ref, wd_ref, sd_ref, bd_ref,
                  w2_ref, b2_ref, o_ref, pad_ref, *, c1):
    P, D, H, W, C = x_ref.shape
    M = D * H * W
    ch = C - c1
    Cd = P * ch
    x2 = jnp.concatenate([x_ref[p][..., c1:] for p in range(P)],
                         axis=-1).reshape(M, Cd)
    xp = jnp.concatenate([x_ref[p][..., :c1] for p in range(P)],
                         axis=-1).reshape(M, P * c1)

    y = jnp.dot(x2.astype(jnp.bfloat16), w1_ref[...],
                preferred_element_type=jnp.float32)
    y = jnp.maximum(y + b1_ref[...], 0.0)

    pad_ref[0:1] = jnp.zeros((1, H + 2, W + 2, Cd), jnp.float32)
    pad_ref[D + 1:D + 2] = jnp.zeros((1, H + 2, W + 2, Cd), jnp.float32)
    pad_ref[:, 0:1] = jnp.zeros((D + 2, 1, W + 2, Cd), jnp.float32)
    pad_ref[:, H + 1:H + 2] = jnp.zeros((D + 2, 1, W + 2, Cd), jnp.float32)
    pad_ref[:, :, 0:1] = jnp.zeros((D + 2, H + 2, 1, Cd), jnp.float32)
    pad_ref[:, :, W + 1:W + 2] = jnp.zeros((D + 2, H + 2, 1, Cd), jnp.float32)
    pad_ref[1:D + 1, 1:H + 1, 1:W + 1, :] = y.reshape(D, H, W, Cd)
    acc = jnp.zeros((D, H, W, Cd), jnp.float32)
    for kd in range(3):
        for kh in range(3):
            for kw in range(3):
                idx = kd * 9 + kh * 3 + kw
                tap = pad_ref[kd:kd + D, kh:kh + H, kw:kw + W, :]
                acc = acc + tap * wd_ref[idx:idx + 1, :]
    z = acc * sd_ref[...] + bd_ref[...]

    u = jnp.concatenate([z.reshape(M, Cd).astype(jnp.bfloat16),
                         xp.astype(jnp.bfloat16)], axis=-1)
    v = jnp.dot(u, w2_ref[...], preferred_element_type=jnp.float32)
    g = v + b2_ref[...]
    odd = (lax.broadcasted_iota(jnp.int32, (1, g.shape[1]), 1) % 2) == 1
    out = jnp.where(odd, jnp.maximum(g, 0.0), g)
    for p in range(P):
        o_ref[p] = out[:, p * C:(p + 1) * C].astype(o_ref.dtype)


def _bcast_spec(a):
    return pl.BlockSpec(a.shape, lambda n: (0,) * a.ndim)


def kernel(x, scatter, w1, s1, b1, wd, sd, bd, w2, s2, b2):
    N, C, D, H, W = x.shape
    c1 = scatter.shape[0]
    cm = w1.shape[1]
    oup = w2.shape[1]
    M = D * H * W
    P = _P
    bf = jnp.bfloat16

    xt = jnp.transpose(x, (0, 2, 3, 4, 1))

    ch = C - c1
    w1s = w1 * s1
    w1b = jnp.zeros((P * ch, P * cm), jnp.float32)
    for p in range(P):
        w1b = w1b.at[p * ch:(p + 1) * ch, p * cm:(p + 1) * cm].set(w1s)
    w1b = w1b.astype(bf)
    b1p = jnp.tile(b1, (1, P))
    wdp = jnp.tile(wd, (1, P))
    sdp = jnp.tile(sd, (1, P))
    bdp = jnp.tile(bd, (1, P))
    even = (jnp.arange(oup) % 2 == 0).astype(jnp.float32)[None, :]
    w2s = w2 * (s2 + even)
    scs = scatter * (s2 + even)
    Wc = jnp.zeros((P * cm + P * c1, P * oup), jnp.float32)
    for p in range(P):
        Wc = Wc.at[p * cm:(p + 1) * cm, p * oup:(p + 1) * oup].set(w2s)
        Wc = Wc.at[P * cm + p * c1:P * cm + (p + 1) * c1,
                   p * oup:(p + 1) * oup].set(scs)
    Wc = Wc.astype(bf)
    b2c = jnp.tile(b2, (1, P))

    args = (xt, w1b, b1p, wdp, sdp, bdp, Wc, b2c)
    in_specs = [pl.BlockSpec((P, D, H, W, C), lambda n: (n, 0, 0, 0, 0))]
    in_specs += [_bcast_spec(a) for a in args[1:]]
    out = pl.pallas_call(
        functools.partial(_block_kernel, c1=c1),
        out_shape=jax.ShapeDtypeStruct((N, M, oup), x.dtype),
        grid=(N // P,),
        in_specs=in_specs,
        out_specs=pl.BlockSpec((P, M, oup), lambda n: (n, 0, 0)),
        scratch_shapes=[pltpu.VMEM((D + 2, H + 2, W + 2, P * cm), jnp.float32)],
        compiler_params=pltpu.CompilerParams(
            dimension_semantics=("parallel",)),
    )(*args)
    out = out.reshape(N, D, H, W, oup)
    return jnp.transpose(out, (0, 4, 1, 2, 3))
